# baseline (speedup 1.0000x reference)
import jax
import jax.numpy as jnp
from jax import lax
from jax.experimental import pallas as pl
from jax.experimental.pallas import tpu as pltpu
from jax.experimental.pallas import tpu_sc as plsc

SEQ, BATCH, VOCAB = 50, 1024, 1000
ROWS = SEQ * BATCH
NC, NS, L = 2, 16, 16
NW = NC * NS
BS = 128
SPB = BATCH // BS
NSLABS = SEQ * SPB
JMAX = -(-NSLABS // NW)
NFULL = NSLABS - NW * (JMAX - 1)


def _slab_coords(sigma):
    s = sigma // SPB
    b0 = (sigma % SPB) * BS
    return s, b0


def _body(in_hbm, out_hbm, idx2d, buf, dsem, isem):
    wid = lax.axis_index("s") * NC + lax.axis_index("c")
    has_last = wid < NFULL

    def idx_copy(j):
        sigma = jnp.minimum(wid + NW * j, NSLABS - 1)
        s, b0 = _slab_coords(sigma)
        return pltpu.make_async_copy(
            in_hbm.at[s, pl.ds(b0, BS)], idx2d.at[j], isem
        )

    for j in range(JMAX - 1):
        idx_copy(j).start()

    @pl.when(has_last)
    def _():
        idx_copy(JMAX - 1).start()

    zeros16 = jnp.zeros((L,), jnp.float32)

    def ztile(vt, carry):
        v0 = vt * 8
        for r in range(8):
            for c in range(BS // L):
                buf[v0 + r, pl.ds(c * L, L)] = zeros16
        return carry

    lax.fori_loop(0, VOCAB // 8, ztile, None)

    for j in range(JMAX - 1):
        idx_copy(j).wait()

    @pl.when(has_last)
    def _():
        idx_copy(JMAX - 1).wait()

    iota = lax.broadcasted_iota(jnp.int32, (L,), 0)
    ones16 = jnp.ones((L,), jnp.float32)

    def scat(j, val):
        for g in range(BS // L):
            rows = idx2d[j, pl.ds(g * L, L)]
            plsc.store_scatter(buf, [rows, g * L + iota], val)

    def slab_copy(j):
        sigma = jnp.minimum(wid + NW * j, NSLABS - 1)
        s, b0 = _slab_coords(sigma)
        return pltpu.make_async_copy(
            buf, out_hbm.at[s, pl.ds(0, VOCAB), pl.ds(b0, BS)], dsem
        )

    for j in range(JMAX):
        def step(j=j):
            if j > 0:
                slab_copy(j - 1).wait()
                scat(j - 1, zeros16)
            scat(j, ones16)
            slab_copy(j).start()

        if j == JMAX - 1:
            pl.when(has_last)(step)
        else:
            step()

    slab_copy(0).wait()


def _onehot_sc(flat_idx):
    mesh = plsc.VectorSubcoreMesh(core_axis_name="c", subcore_axis_name="s")
    return pl.kernel(
        _body,
        mesh=mesh,
        out_type=jax.ShapeDtypeStruct((SEQ, VOCAB, BATCH), jnp.float32),
        scratch_types=[
            pltpu.VMEM((JMAX, BS), jnp.int32),
            pltpu.VMEM((VOCAB, BS), jnp.float32),
            pltpu.SemaphoreType.DMA,
            pltpu.SemaphoreType.DMA,
        ],
        compiler_params=pltpu.CompilerParams(
            use_tc_tiling_on_sc=True, needs_layout_passes=False
        ),
    )(flat_idx)


def kernel(input):
    out = _onehot_sc(input.astype(jnp.int32))
    return jnp.swapaxes(out, 1, 2)


# speedup vs baseline: 1.0004x; 1.0004x over previous
import jax
import jax.numpy as jnp
from jax import lax
from jax.experimental import pallas as pl
from jax.experimental.pallas import tpu as pltpu
from jax.experimental.pallas import tpu_sc as plsc

SEQ, BATCH, VOCAB = 50, 1024, 1000
ROWS = SEQ * BATCH
NC, NS, L = 2, 16, 16
NW = NC * NS
BS = 128
SPB = BATCH // BS
NSLABS = SEQ * SPB
JMAX = -(-NSLABS // NW)
NFULL = NSLABS - NW * (JMAX - 1)


def _slab_coords(sigma):
    s = sigma // SPB
    b0 = (sigma % SPB) * BS
    return s, b0


def _body(in_hbm, out_hbm, idx2d, buf, dsem, isem):
    wid = lax.axis_index("s") * NC + lax.axis_index("c")
    has_last = wid < NFULL

    def idx_copy(j):
        sigma = jnp.minimum(wid + NW * j, NSLABS - 1)
        s, b0 = _slab_coords(sigma)
        return pltpu.make_async_copy(
            in_hbm.at[s, pl.ds(b0, BS)], idx2d.at[j], isem
        )

    for j in range(JMAX - 1):
        idx_copy(j).start()

    @pl.when(has_last)
    def _():
        idx_copy(JMAX - 1).start()

    zeros16 = jnp.zeros((L,), jnp.float32)

    def ztile(vt, carry):
        v0 = vt * 8
        for r in range(8):
            for c in range(BS // L):
                buf[v0 + r, pl.ds(c * L, L)] = zeros16
        return carry

    lax.fori_loop(0, VOCAB // 8, ztile, None)

    for j in range(JMAX - 1):
        idx_copy(j).wait()

    @pl.when(has_last)
    def _():
        idx_copy(JMAX - 1).wait()

    iota = lax.broadcasted_iota(jnp.int32, (L,), 0)
    ones16 = jnp.ones((L,), jnp.float32)

    def scat(j, val):
        for g in range(BS // L):
            rows = idx2d[j, pl.ds(g * L, L)]
            plsc.store_scatter(buf, [rows, g * L + iota], val)

    VH0 = 504

    def slab_copy(j, half):
        sigma = jnp.minimum(wid + NW * j, NSLABS - 1)
        s, b0 = _slab_coords(sigma)
        v0, vn = (0, VH0) if half == 0 else (VH0, VOCAB - VH0)
        return pltpu.make_async_copy(
            buf.at[pl.ds(v0, vn)],
            out_hbm.at[s, pl.ds(v0, vn), pl.ds(b0, BS)],
            dsem,
        )

    for j in range(JMAX):
        def step(j=j):
            if j > 0:
                slab_copy(j - 1, 0).wait()
                slab_copy(j - 1, 1).wait()
                scat(j - 1, zeros16)
            scat(j, ones16)
            slab_copy(j, 0).start()
            slab_copy(j, 1).start()

        if j == JMAX - 1:
            pl.when(has_last)(step)
        else:
            step()

    slab_copy(0, 0).wait()
    slab_copy(0, 1).wait()


def _onehot_sc(flat_idx):
    mesh = plsc.VectorSubcoreMesh(core_axis_name="c", subcore_axis_name="s")
    return pl.kernel(
        _body,
        mesh=mesh,
        out_type=jax.ShapeDtypeStruct((SEQ, VOCAB, BATCH), jnp.float32),
        scratch_types=[
            pltpu.VMEM((JMAX, BS), jnp.int32),
            pltpu.VMEM((VOCAB, BS), jnp.float32),
            pltpu.SemaphoreType.DMA,
            pltpu.SemaphoreType.DMA,
        ],
        compiler_params=pltpu.CompilerParams(
            use_tc_tiling_on_sc=True, needs_layout_passes=False
        ),
    )(flat_idx)


def kernel(input):
    out = _onehot_sc(input.astype(jnp.int32))
    return jnp.swapaxes(out, 1, 2)


# speedup vs baseline: 1.0024x; 1.0020x over previous
import jax
import jax.numpy as jnp
from jax import lax
from jax.experimental import pallas as pl
from jax.experimental.pallas import tpu as pltpu
from jax.experimental.pallas import tpu_sc as plsc

SEQ, BATCH, VOCAB = 50, 1024, 1000
ROWS = SEQ * BATCH
NC, NS, L = 2, 16, 16
NW = NC * NS
BS = 128
SPB = BATCH // BS
NSLABS = SEQ * SPB
JMAX = -(-NSLABS // NW)
NFULL = NSLABS - NW * (JMAX - 1)


def _slab_coords(sigma):
    s = sigma // SPB
    b0 = (sigma % SPB) * BS
    return s, b0


def _body(in_hbm, out_hbm, idx2d, buf, dsem, isem):
    wid = lax.axis_index("s") * NC + lax.axis_index("c")
    has_last = wid < NFULL

    def idx_copy(j):
        sigma = jnp.minimum(wid + NW * j, NSLABS - 1)
        s, b0 = _slab_coords(sigma)
        return pltpu.make_async_copy(
            in_hbm.at[s, pl.ds(b0, BS)], idx2d.at[j], isem
        )

    for j in range(JMAX - 1):
        idx_copy(j).start()

    @pl.when(has_last)
    def _():
        idx_copy(JMAX - 1).start()

    zeros16 = jnp.zeros((L,), jnp.float32)

    def ztile(vt, carry):
        v0 = vt * 8
        for r in range(8):
            for c in range(BS // L):
                buf[v0 + r, pl.ds(c * L, L)] = zeros16
        return carry

    lax.fori_loop(0, VOCAB // 8, ztile, None)

    for j in range(JMAX - 1):
        idx_copy(j).wait()

    @pl.when(has_last)
    def _():
        idx_copy(JMAX - 1).wait()

    iota = lax.broadcasted_iota(jnp.int32, (L,), 0)
    ones16 = jnp.ones((L,), jnp.float32)

    def scat(j, val):
        for g in range(BS // L):
            rows = idx2d[j, pl.ds(g * L, L)]
            plsc.store_scatter(buf, [rows, g * L + iota], val)

    def slab_copy(j):
        sigma = jnp.minimum(wid + NW * j, NSLABS - 1)
        s, b0 = _slab_coords(sigma)
        return pltpu.make_async_copy(
            buf, out_hbm.at[s, pl.ds(0, VOCAB), pl.ds(b0, BS)], dsem
        )

    for j in range(JMAX):
        def step(j=j):
            if j > 0:
                slab_copy(j - 1).wait()
                scat(j - 1, zeros16)
            scat(j, ones16)
            slab_copy(j).start()

        if j == JMAX - 1:
            pl.when(has_last)(step)
        else:
            step()

    slab_copy(0).wait()


def _onehot_sc(flat_idx):
    mesh = plsc.VectorSubcoreMesh(core_axis_name="c", subcore_axis_name="s")
    return pl.kernel(
        _body,
        mesh=mesh,
        out_type=jax.ShapeDtypeStruct((SEQ, VOCAB, BATCH), jnp.float32),
        scratch_types=[
            pltpu.VMEM((JMAX, BS), jnp.int32),
            pltpu.VMEM((VOCAB, BS), jnp.float32),
            pltpu.SemaphoreType.DMA,
            pltpu.SemaphoreType.DMA,
        ],
        compiler_params=pltpu.CompilerParams(
            use_tc_tiling_on_sc=True, needs_layout_passes=False
        ),
    )(flat_idx)


def kernel(input):
    out = _onehot_sc(input.astype(jnp.int32))
    return jnp.swapaxes(out, 1, 2)


# speedup vs baseline: 1.0032x; 1.0008x over previous
import jax
import jax.numpy as jnp
from jax import lax
from jax.experimental import pallas as pl
from jax.experimental.pallas import tpu as pltpu
from jax.experimental.pallas import tpu_sc as plsc

SEQ, BATCH, VOCAB = 50, 1024, 1000
ROWS = SEQ * BATCH
NC, NS, L = 2, 16, 16
NW = NC * NS
BS = 128
SPB = BATCH // BS
NSLABS = SEQ * SPB
JMAX = -(-NSLABS // NW)
NFULL = NSLABS - NW * (JMAX - 1)


def _slab_coords(sigma):
    s = sigma // SPB
    b0 = (sigma % SPB) * BS
    return s, b0


def _body(in_hbm, out_hbm, idx2d, buf, dsem, isem):
    wid = lax.axis_index("s") * NC + lax.axis_index("c")
    has_last = wid < NFULL

    def idx_copy(j):
        sigma = jnp.minimum(wid + NW * j, NSLABS - 1)
        s, b0 = _slab_coords(sigma)
        return pltpu.make_async_copy(
            in_hbm.at[pl.ds(s * BATCH + b0, BS)], idx2d.at[j], isem
        )

    for j in range(JMAX - 1):
        idx_copy(j).start()

    @pl.when(has_last)
    def _():
        idx_copy(JMAX - 1).start()

    zeros16 = jnp.zeros((L,), jnp.float32)

    def ztile(vt, carry):
        v0 = vt * 8
        for r in range(8):
            for c in range(BS // L):
                buf[v0 + r, pl.ds(c * L, L)] = zeros16
        return carry

    lax.fori_loop(0, VOCAB // 8, ztile, None)

    for j in range(JMAX - 1):
        idx_copy(j).wait()

    @pl.when(has_last)
    def _():
        idx_copy(JMAX - 1).wait()

    iota = lax.broadcasted_iota(jnp.int32, (L,), 0)
    ones16 = jnp.ones((L,), jnp.float32)

    def scat(j, val):
        for g in range(BS // L):
            rows = idx2d[j, pl.ds(g * L, L)]
            plsc.store_scatter(buf, [rows, g * L + iota], val)

    def slab_copy(j):
        sigma = jnp.minimum(wid + NW * j, NSLABS - 1)
        s, b0 = _slab_coords(sigma)
        return pltpu.make_async_copy(
            buf, out_hbm.at[s, pl.ds(0, VOCAB), pl.ds(b0, BS)], dsem
        )

    for j in range(JMAX):
        def step(j=j):
            if j > 0:
                slab_copy(j - 1).wait()
                scat(j - 1, zeros16)
            scat(j, ones16)
            slab_copy(j).start()

        if j == JMAX - 1:
            pl.when(has_last)(step)
        else:
            step()

    slab_copy(0).wait()


def _onehot_sc(flat_idx):
    mesh = plsc.VectorSubcoreMesh(core_axis_name="c", subcore_axis_name="s")
    return pl.kernel(
        _body,
        mesh=mesh,
        out_type=jax.ShapeDtypeStruct((SEQ, VOCAB, BATCH), jnp.float32),
        scratch_types=[
            pltpu.VMEM((JMAX, BS), jnp.int32),
            pltpu.VMEM((VOCAB, BS), jnp.float32),
            pltpu.SemaphoreType.DMA,
            pltpu.SemaphoreType.DMA,
        ],
        compiler_params=pltpu.CompilerParams(
            use_tc_tiling_on_sc=True, needs_layout_passes=False
        ),
    )(flat_idx)


def kernel(input):
    flat_idx = input.reshape(ROWS).astype(jnp.int32)
    out = _onehot_sc(flat_idx)
    return jnp.swapaxes(out, 1, 2)
